# baseline (speedup 1.0000x reference)
import functools

import jax
import jax.numpy as jnp
from jax import lax
from jax.experimental import pallas as pl
from jax.experimental.pallas import tpu as pltpu
from jax.experimental.pallas import tpu_sc as plsc

N = 10000
E = 160000
NTI = 16
NWIN = 125
WB = 80
NP = 10240
RPT = NP // NTI
NDEG = NP
DSTR = NDEG // NTI
G = 5
NG = NWIN // G
F0 = 32
LAYERS = 5
NQ = 8
FH = 320
R = 1000


def _sc_mesh():
    return plsc.VectorSubcoreMesh(core_axis_name="c", subcore_axis_name="s")


_SC_PARAMS = pltpu.CompilerParams(use_tc_tiling_on_sc=False)


@functools.cache
def _get_sc_degree():
    @functools.partial(
        pl.kernel,
        mesh=_sc_mesh(),
        compiler_params=_SC_PARAMS,
        out_type=jax.ShapeDtypeStruct((NDEG,), jnp.float32),
        scratch_types=[
            pltpu.VMEM((NWIN, WB), jnp.int32),
            pltpu.VMEM((WB,), jnp.float32),
            pltpu.VMEM_SHARED((NDEG,), jnp.float32),
            pltpu.SemaphoreType.DMA,
        ],
    )
    def _sc_degree(dst_h, ones_h, out_h, dst_v, ones_v, acc_sh, dsem):
        c = lax.axis_index("c")
        s = lax.axis_index("s")

        @pl.when(c == 0)
        def _():
            pltpu.sync_copy(ones_h.at[pl.ds(s * DSTR, DSTR)],
                            acc_sh.at[pl.ds(s * DSTR, DSTR)])
            pltpu.sync_copy(ones_h.at[pl.ds(0, WB)], ones_v)
            pltpu.sync_copy(dst_h.at[s], dst_v)
            plsc.subcore_barrier()

            def fire(g):
                for j in range(G):
                    pltpu.async_copy(ones_v, acc_sh.at[dst_v.at[g * G + j]],
                                     dsem, add=True)

            def drain():
                for _ in range(G):
                    pltpu.make_async_copy(ones_v, acc_sh.at[dst_v.at[0]],
                                          dsem).wait()

            fire(0)

            def body(g, carry):
                @pl.when(g < NG - 1)
                def _():
                    fire(g + 1)

                drain()
                return carry

            lax.fori_loop(0, NG, body, 0)
            plsc.subcore_barrier()
            pltpu.sync_copy(acc_sh.at[pl.ds(s * DSTR, DSTR)],
                            out_h.at[pl.ds(s * DSTR, DSTR)])

    return _sc_degree


@functools.cache
def _make_sc_agg(f2, nphase):
    nz = 2 * nphase

    @functools.partial(
        pl.kernel,
        mesh=_sc_mesh(),
        compiler_params=_SC_PARAMS,
        out_type=tuple(jax.ShapeDtypeStruct((NP, f2), jnp.float32)
                       for _ in range(nz)),
        scratch_types=[
            pltpu.VMEM((NWIN, WB), jnp.int32),
            pltpu.VMEM((NWIN, WB), jnp.int32),
            pltpu.VMEM((2, G, WB, f2), jnp.float32),
            pltpu.VMEM_SHARED((NP, f2), jnp.float32),
            pltpu.SemaphoreType.DMA,
            pltpu.SemaphoreType.DMA,
        ],
    )
    def agg(*refs):
        z_hs = refs[:nz]
        src_h, dst_h, zeros_h = refs[nz:nz + 3]
        o_hs = refs[nz + 3:nz + 3 + nz]
        src_v, dst_v, rows_v, acc_sh, gsem, ssem = refs[nz + 3 + nz:]
        c = lax.axis_index("c")
        s = lax.axis_index("s")
        pltpu.sync_copy(src_h.at[s], src_v)
        pltpu.sync_copy(dst_h.at[s], dst_v)

        def phase(za_h, zb_h, oa_h, ob_h):
            pltpu.sync_copy(zeros_h.at[pl.ds(s * RPT, RPT)],
                            acc_sh.at[pl.ds(s * RPT, RPT)])
            plsc.subcore_barrier()

            def gathers(g, half):
                for j in range(G):
                    iv = src_v.at[g * G + j]

                    @pl.when(c == 0)
                    def _():
                        pltpu.async_copy(za_h.at[iv], rows_v.at[half, j],
                                         gsem)

                    @pl.when(c == 1)
                    def _():
                        pltpu.async_copy(zb_h.at[iv], rows_v.at[half, j],
                                         gsem)

            def wait_gathers(half):
                for j in range(G):
                    pltpu.make_async_copy(za_h.at[src_v.at[0]],
                                          rows_v.at[half, j], gsem).wait()

            def scatters(g, half):
                for j in range(G):
                    pltpu.async_copy(rows_v.at[half, j],
                                     acc_sh.at[dst_v.at[g * G + j]], ssem,
                                     add=True)

            def wait_scatters(half):
                for j in range(G):
                    pltpu.make_async_copy(rows_v.at[half, j],
                                          acc_sh.at[dst_v.at[0]],
                                          ssem).wait()

            gathers(0, 0)

            def body(gp, carry):
                for par in (0, 1):
                    g = 2 * gp + par
                    wait_gathers(par)
                    if par == 1:
                        wait_scatters(0)
                    else:
                        @pl.when(gp >= 1)
                        def _():
                            wait_scatters(1)

                    gathers(g + 1, 1 - par)
                    scatters(g, par)
                return carry

            lax.fori_loop(0, (NG - 1) // 2, body, 0)
            wait_gathers(0)
            wait_scatters(1)
            scatters(NG - 1, 0)
            wait_scatters(0)
            plsc.subcore_barrier()

            @pl.when(c == 0)
            def _():
                pltpu.sync_copy(acc_sh.at[pl.ds(s * RPT, RPT)],
                                oa_h.at[pl.ds(s * RPT, RPT)])

            @pl.when(c == 1)
            def _():
                pltpu.sync_copy(acc_sh.at[pl.ds(s * RPT, RPT)],
                                ob_h.at[pl.ds(s * RPT, RPT)])

        for p in range(nphase):
            phase(z_hs[p], z_hs[nphase + p], o_hs[p], o_hs[nphase + p])

    return agg


def _embed_lo_body(ea_ref, we_ref, be_ref, deg_ref, attr_ref, zlo_ref, zhi_ref):
    e = (ea_ref[:, 0:1] * we_ref[0:1, :]
         + ea_ref[:, 1:2] * we_ref[1:2, :]
         + ea_ref[:, 2:3] * we_ref[2:3, :]
         + be_ref[...])
    dinv = lax.rsqrt(deg_ref[...])
    z = e * dinv
    attr_ref[...] = e
    zlo_ref[...] = z[:, : F0 // 2]
    zhi_ref[...] = z[:, F0 // 2:]


def _mlpall_body(ea_ref, we_ref, be_ref, w0_ref, wg_ref, bs_ref, o_ref):
    h = (ea_ref[:, 0:1] * we_ref[0:1, :]
         + ea_ref[:, 1:2] * we_ref[1:2, :]
         + ea_ref[:, 2:3] * we_ref[2:3, :]
         + be_ref[...])
    h = jnp.dot(h.astype(jnp.bfloat16), w0_ref[...],
                preferred_element_type=jnp.float32)
    h = jnp.maximum(h + bs_ref[0:1, :], 0.0)
    for l in range(LAYERS - 1):
        h = jnp.dot(h.astype(jnp.bfloat16), wg_ref[l],
                    preferred_element_type=jnp.float32)
        h = jnp.maximum(h + bs_ref[l + 1:l + 2, :], 0.0)

    @pl.when(pl.program_id(0) == 0)
    def _():
        o_ref[...] = jnp.zeros_like(o_ref)

    o_ref[...] += jnp.sum(h, axis=0, keepdims=True)


def _mlp_body(a_ref, w_ref, b_ref, o_ref):
    h = jnp.dot(a_ref[...].astype(jnp.bfloat16),
                w_ref[...].astype(jnp.bfloat16),
                preferred_element_type=jnp.float32)
    o_ref[...] = jnp.maximum(h + b_ref[...], 0.0)


def _mix_body(*refs):
    nag = len(refs) - 13
    a_ref = refs[0]
    ag_refs = refs[1:1 + nag]
    deg_ref, w_ref, b_ref = refs[1 + nag:4 + nag]
    o_ref = refs[4 + nag]
    zq_refs = refs[5 + nag:]
    assert len(zq_refs) == NQ
    dinv = lax.rsqrt(deg_ref[...])
    ag = jnp.concatenate([r[...] for r in ag_refs], axis=1)
    u = ag * dinv + a_ref[...] * (dinv * dinv)
    h = jnp.dot(u.astype(jnp.bfloat16), w_ref[...].astype(jnp.bfloat16),
                preferred_element_type=jnp.float32)
    o = jnp.maximum(h + b_ref[...], 0.0)
    o_ref[...] = o
    z = o * dinv
    q = FH // NQ
    for i, zr in enumerate(zq_refs):
        zr[...] = z[:, i * q:(i + 1) * q]


def _colsum_body(a_ref, o_ref):
    @pl.when(pl.program_id(0) == 0)
    def _():
        o_ref[...] = jnp.zeros_like(o_ref)

    o_ref[...] += jnp.sum(a_ref[...], axis=0, keepdims=True)


def _head_body(*refs):
    sum_refs = refs[:-5]
    w1_ref, b1_ref, w2_ref, b2_ref, o_ref = refs[-5:]
    tot = sum_refs[0][...]
    for r in sum_refs[1:]:
        tot = tot + r[...]
    m = tot * (1.0 / float(E))
    g = jnp.dot(m, w1_ref[...], preferred_element_type=jnp.float32)
    g = jnp.maximum(g + b1_ref[...], 0.0)
    o_ref[...] = jnp.dot(g, w2_ref[...],
                         preferred_element_type=jnp.float32) + b2_ref[...]


def _rows(r, f):
    return pl.BlockSpec((r, f), lambda i: (i, 0))


def _whole(shape):
    return pl.BlockSpec(shape, lambda i: tuple(0 for _ in shape))


def _embed_lo(ea, we, be, deg):
    return pl.pallas_call(
        _embed_lo_body,
        grid=(N // R,),
        in_specs=[_rows(R, 3), _whole((3, F0)), _whole((1, F0)), _rows(R, 1)],
        out_specs=[_rows(R, F0), _rows(R, F0 // 2), _rows(R, F0 // 2)],
        out_shape=[
            jax.ShapeDtypeStruct((N, F0), jnp.float32),
            jax.ShapeDtypeStruct((N, F0 // 2), jnp.float32),
            jax.ShapeDtypeStruct((N, F0 // 2), jnp.float32),
        ],
    )(ea, we, be, deg)


def _mlpall(ea, we, be, w0, wg, bs):
    m = ea.shape[0]
    return pl.pallas_call(
        _mlpall_body,
        grid=(m // R,),
        in_specs=[_rows(R, 3), _whole((3, F0)), _whole((1, F0)),
                  _whole((F0, FH)), _whole((LAYERS - 1, FH, FH)),
                  _whole((LAYERS, FH))],
        out_specs=_whole((1, FH)),
        out_shape=jax.ShapeDtypeStruct((1, FH), jnp.float32),
    )(ea, we, be, w0, wg, bs)


def _mlp(a, w, b):
    m, f = a.shape
    fo = w.shape[1]
    return pl.pallas_call(
        _mlp_body,
        grid=(m // R,),
        in_specs=[_rows(R, f), _whole((f, fo)), _whole((1, fo))],
        out_specs=_rows(R, fo),
        out_shape=jax.ShapeDtypeStruct((m, fo), jnp.float32),
    )(a, w, b)


def _mix(a, ags, deg, w, b):
    f = a.shape[1]
    fo = w.shape[1]
    q = fo // NQ
    return pl.pallas_call(
        _mix_body,
        grid=(N // R,),
        in_specs=([_rows(R, f)] + [_rows(R, g.shape[1]) for g in ags]
                  + [_rows(R, 1), _whole((f, fo)), _whole((1, fo))]),
        out_specs=[_rows(R, fo)] + [_rows(R, q)] * NQ,
        out_shape=([jax.ShapeDtypeStruct((N, fo), jnp.float32)]
                   + [jax.ShapeDtypeStruct((N, q), jnp.float32)] * NQ),
    )(a, *ags, deg, w, b)


def _colsum(a):
    m, f = a.shape
    return pl.pallas_call(
        _colsum_body,
        grid=(m // R,),
        in_specs=[_rows(R, f)],
        out_specs=_whole((1, f)),
        out_shape=jax.ShapeDtypeStruct((1, f), jnp.float32),
    )(a)


def _head(sums, w1, b1, w2, b2):
    return pl.pallas_call(
        _head_body,
        grid=(1,),
        in_specs=([_whole((1, FH))] * len(sums)
                  + [_whole((FH, 32)), _whole((1, 32)), _whole((32, 2)),
                     _whole((1, 2))]),
        out_specs=_whole((1, 2)),
        out_shape=jax.ShapeDtypeStruct((1, 2), jnp.float32),
    )(*sums, w1, b1, w2, b2)


def kernel(x, edge_index, edge_attr, batch, Wn, bn, We, be, Wg0, bg0, Wg, bg,
           W1, b1, W2, b2):
    src3 = edge_index[0].reshape(NTI, NWIN, WB)
    dst3 = edge_index[1].reshape(NTI, NWIN, WB)

    ones_d = jnp.ones((NDEG,), jnp.float32)
    zeros0 = jnp.zeros((NP, F0 // 2), jnp.float32)
    zerosh = jnp.zeros((NP, FH // NQ), jnp.float32)

    deg = _get_sc_degree()(dst3, ones_d)[:N].reshape(N, 1)

    wg0 = jnp.pad(Wg0, ((0, 0), (0, FH - Wg0.shape[1])))
    bg0p = jnp.pad(bg0, (0, FH - bg0.shape[0])).reshape(1, FH)
    wgs = [jnp.pad(Wg[i], ((0, FH - Wg.shape[1]), (0, FH - Wg.shape[2])))
           for i in range(Wg.shape[0])]
    bgs = [jnp.pad(bg[i], (0, FH - bg.shape[1])).reshape(1, FH)
           for i in range(Wg.shape[0])]
    w1p = jnp.pad(W1, ((0, FH - W1.shape[0]), (0, 0)))
    b1p = b1.reshape(1, 32)
    b2p = b2.reshape(1, 2)

    wep = jnp.pad(We, ((0, 0), (0, 0)))
    bep = be.reshape(1, F0)

    attr_lo, zlo, zhi = _embed_lo(edge_attr[:N], wep, bep, deg)
    wg_bf = jnp.stack(wgs[0:]).astype(jnp.bfloat16)
    bs_all = jnp.concatenate([bg0p] + bgs, axis=0)
    w0_bf = wg0.astype(jnp.bfloat16)
    shis = []
    splits = [(N, 47000), (57000, 37000), (94000, 33000), (127000, 33000)]
    for start, count in splits:
        shis.append(_mlpall(edge_attr[start:start + count], wep, bep,
                            w0_bf, wg_bf, bs_all))

    ws = [wg0] + wgs
    bs = [bg0p] + bgs
    aglo, aghi = _make_sc_agg(F0 // 2, 1)(zlo, zhi, src3, dst3, zeros0)
    out = _mix(attr_lo, [aglo, aghi], deg, ws[0], bs[0])
    attr_lo, zq = out[0], list(out[1:])
    for l in range(1, len(ws)):
        ags = list(_make_sc_agg(FH // NQ, NQ // 2)(*zq, src3, dst3, zerosh))
        out = _mix(attr_lo, ags, deg, ws[l], bs[l])
        attr_lo, zq = out[0], list(out[1:])

    slo = _colsum(attr_lo)
    return _head([slo] + shis, w1p, b1p, W2, b2p)


# speedup vs baseline: 1.2188x; 1.2188x over previous
import functools

import jax
import jax.numpy as jnp
from jax import lax
from jax.experimental import pallas as pl
from jax.experimental.pallas import tpu as pltpu
from jax.experimental.pallas import tpu_sc as plsc

N = 10000
E = 160000
NTI = 16
NWIN = 125
WB = 80
NP = 10240
RPT = NP // NTI
NDEG = NP
DSTR = NDEG // NTI
G = 5
NG = NWIN // G
F0 = 32
LAYERS = 5
NQ = 8
FH = 320
R = 1000


def _sc_mesh():
    return plsc.VectorSubcoreMesh(core_axis_name="c", subcore_axis_name="s")


_SC_PARAMS = pltpu.CompilerParams(use_tc_tiling_on_sc=False)


@functools.cache
def _get_sc_degree():
    @functools.partial(
        pl.kernel,
        mesh=_sc_mesh(),
        compiler_params=_SC_PARAMS,
        out_type=jax.ShapeDtypeStruct((NDEG,), jnp.float32),
        scratch_types=[
            pltpu.VMEM((NWIN, WB), jnp.int32),
            pltpu.VMEM((WB,), jnp.float32),
            pltpu.VMEM_SHARED((NDEG,), jnp.float32),
            pltpu.SemaphoreType.DMA,
        ],
    )
    def _sc_degree(dst_h, ones_h, out_h, dst_v, ones_v, acc_sh, dsem):
        c = lax.axis_index("c")
        s = lax.axis_index("s")

        @pl.when(c == 0)
        def _():
            pltpu.sync_copy(ones_h.at[pl.ds(s * DSTR, DSTR)],
                            acc_sh.at[pl.ds(s * DSTR, DSTR)])
            pltpu.sync_copy(ones_h.at[pl.ds(0, WB)], ones_v)
            pltpu.sync_copy(dst_h.at[s], dst_v)
            plsc.subcore_barrier()

            def fire(g):
                for j in range(G):
                    pltpu.async_copy(ones_v, acc_sh.at[dst_v.at[g * G + j]],
                                     dsem, add=True)

            def drain():
                for _ in range(G):
                    pltpu.make_async_copy(ones_v, acc_sh.at[dst_v.at[0]],
                                          dsem).wait()

            fire(0)

            def body(g, carry):
                @pl.when(g < NG - 1)
                def _():
                    fire(g + 1)

                drain()
                return carry

            lax.fori_loop(0, NG, body, 0)
            plsc.subcore_barrier()
            pltpu.sync_copy(acc_sh.at[pl.ds(s * DSTR, DSTR)],
                            out_h.at[pl.ds(s * DSTR, DSTR)])

    return _sc_degree


@functools.cache
def _make_sc_agg(f2, nphase, wb, nwin):
    half = nphase * f2
    fw = 2 * half
    nz = 2 * nphase
    ng = nwin // G

    @functools.partial(
        pl.kernel,
        mesh=_sc_mesh(),
        compiler_params=_SC_PARAMS,
        cost_estimate=pl.CostEstimate(flops=0, transcendentals=0,
                                      bytes_accessed=2 * E * fw * 4),
        out_type=jax.ShapeDtypeStruct((NP, fw), jnp.float32),
        scratch_types=[
            pltpu.VMEM((nwin, wb), jnp.int32),
            pltpu.VMEM((nwin, wb), jnp.int32),
            pltpu.VMEM((2, G, wb, f2), jnp.float32),
            pltpu.VMEM_SHARED((NP, f2), jnp.float32),
            pltpu.SemaphoreType.DMA,
            pltpu.SemaphoreType.DMA,
        ],
    )
    def agg(*refs):
        z_hs = refs[:nz]
        src_h, dst_h, zeros_h, out_h = refs[nz:nz + 4]
        src_v, dst_v, rows_v, acc_sh, gsem, ssem = refs[nz + 4:]
        c = lax.axis_index("c")
        s = lax.axis_index("s")
        pltpu.sync_copy(src_h.at[s], src_v)
        pltpu.sync_copy(dst_h.at[s], dst_v)

        def phase(p):
            za_h = z_hs[p]
            zb_h = z_hs[nphase + p]
            pltpu.sync_copy(zeros_h.at[pl.ds(s * RPT, RPT)],
                            acc_sh.at[pl.ds(s * RPT, RPT)])
            plsc.subcore_barrier()

            def gathers(g, hbuf):
                for j in range(G):
                    iv = src_v.at[g * G + j]

                    @pl.when(c == 0)
                    def _():
                        pltpu.async_copy(za_h.at[iv], rows_v.at[hbuf, j],
                                         gsem)

                    @pl.when(c == 1)
                    def _():
                        pltpu.async_copy(zb_h.at[iv], rows_v.at[hbuf, j],
                                         gsem)

            def wait_gathers(hbuf):
                for j in range(G):
                    pltpu.make_async_copy(za_h.at[src_v.at[0]],
                                          rows_v.at[hbuf, j], gsem).wait()

            def scatters(g, hbuf):
                for j in range(G):
                    pltpu.async_copy(rows_v.at[hbuf, j],
                                     acc_sh.at[dst_v.at[g * G + j]], ssem,
                                     add=True)

            def wait_scatters(hbuf):
                for j in range(G):
                    pltpu.make_async_copy(rows_v.at[hbuf, j],
                                          acc_sh.at[dst_v.at[0]],
                                          ssem).wait()

            gathers(0, 0)

            def body(gp, carry):
                for par in (0, 1):
                    g = 2 * gp + par
                    wait_gathers(par)
                    if par == 1:
                        wait_scatters(0)
                    else:
                        @pl.when(gp >= 1)
                        def _():
                            wait_scatters(1)

                    @pl.when(g + 1 < ng)
                    def _():
                        gathers(g + 1, 1 - par)

                    scatters(g, par)
                return carry

            lax.fori_loop(0, ng // 2, body, 0)
            if ng % 2 == 1:
                wait_gathers(0)
                wait_scatters(1)
                scatters(ng - 1, 0)
                wait_scatters(0)
            else:
                wait_scatters(1)
            plsc.subcore_barrier()

            @pl.when(c == 0)
            def _():
                pltpu.sync_copy(
                    acc_sh.at[pl.ds(s * RPT, RPT)],
                    out_h.at[pl.ds(s * RPT, RPT), pl.ds(p * f2, f2)])

            @pl.when(c == 1)
            def _():
                pltpu.sync_copy(
                    acc_sh.at[pl.ds(s * RPT, RPT)],
                    out_h.at[pl.ds(s * RPT, RPT), pl.ds(half + p * f2, f2)])

        for p in range(nphase):
            phase(p)

    return agg


def _embed_lo_body(ea_ref, we_ref, be_ref, deg_ref, attr_ref, zlo_ref, zhi_ref):
    e = (ea_ref[:, 0:1] * we_ref[0:1, :]
         + ea_ref[:, 1:2] * we_ref[1:2, :]
         + ea_ref[:, 2:3] * we_ref[2:3, :]
         + be_ref[...])
    dinv = lax.rsqrt(deg_ref[...])
    attr_ref[...] = e
    z = e * dinv
    zlo_ref[...] = z[:, : F0 // 2]
    zhi_ref[...] = z[:, F0 // 2:]


def _mlpall_body(ea_ref, we_ref, be_ref, w0_ref, wg_ref, bs_ref, o_ref):
    h = (ea_ref[:, 0:1] * we_ref[0:1, :]
         + ea_ref[:, 1:2] * we_ref[1:2, :]
         + ea_ref[:, 2:3] * we_ref[2:3, :]
         + be_ref[...])
    h = jnp.dot(h.astype(jnp.bfloat16), w0_ref[...],
                preferred_element_type=jnp.float32)
    h = jnp.maximum(h + bs_ref[0:1, :], 0.0)
    for l in range(LAYERS - 1):
        h = jnp.dot(h.astype(jnp.bfloat16), wg_ref[l],
                    preferred_element_type=jnp.float32)
        h = jnp.maximum(h + bs_ref[l + 1:l + 2, :], 0.0)

    @pl.when(pl.program_id(0) == 0)
    def _():
        o_ref[...] = jnp.zeros_like(o_ref)

    o_ref[...] += jnp.sum(h, axis=0, keepdims=True)


def _mlp_body(a_ref, w_ref, b_ref, o_ref):
    h = jnp.dot(a_ref[...].astype(jnp.bfloat16),
                w_ref[...].astype(jnp.bfloat16),
                preferred_element_type=jnp.float32)
    o_ref[...] = jnp.maximum(h + b_ref[...], 0.0)


def _mix_body(a_ref, ag_ref, deg_ref, w_ref, b_ref, o_ref, *z_refs):
    dinv = lax.rsqrt(deg_ref[...])
    u = ag_ref[...] * dinv + a_ref[...] * (dinv * dinv)
    h = jnp.dot(u.astype(jnp.bfloat16), w_ref[...].astype(jnp.bfloat16),
                preferred_element_type=jnp.float32)
    o = jnp.maximum(h + b_ref[...], 0.0)
    o_ref[...] = o
    z = o * dinv
    q = FH // 4
    for i, zr in enumerate(z_refs):
        zr[...] = z[:, i * q:(i + 1) * q]


def _colsum_body(a_ref, o_ref):
    @pl.when(pl.program_id(0) == 0)
    def _():
        o_ref[...] = jnp.zeros_like(o_ref)

    o_ref[...] += jnp.sum(a_ref[...], axis=0, keepdims=True)


def _head_body(*refs):
    sum_refs = refs[:-5]
    w1_ref, b1_ref, w2_ref, b2_ref, o_ref = refs[-5:]
    tot = sum_refs[0][...]
    for r in sum_refs[1:]:
        tot = tot + r[...]
    m = tot * (1.0 / float(E))
    g = jnp.dot(m, w1_ref[...], preferred_element_type=jnp.float32)
    g = jnp.maximum(g + b1_ref[...], 0.0)
    o_ref[...] = jnp.dot(g, w2_ref[...],
                         preferred_element_type=jnp.float32) + b2_ref[...]


def _rows(r, f):
    return pl.BlockSpec((r, f), lambda i: (i, 0))


def _whole(shape):
    return pl.BlockSpec(shape, lambda i: tuple(0 for _ in shape))


def _embed_lo(ea, we, be, deg):
    return pl.pallas_call(
        _embed_lo_body,
        grid=(N // R,),
        in_specs=[_rows(R, 3), _whole((3, F0)), _whole((1, F0)), _rows(R, 1)],
        out_specs=[_rows(R, F0), _rows(R, F0 // 2), _rows(R, F0 // 2)],
        out_shape=[
            jax.ShapeDtypeStruct((N, F0), jnp.float32),
            jax.ShapeDtypeStruct((N, F0 // 2), jnp.float32),
            jax.ShapeDtypeStruct((N, F0 // 2), jnp.float32),
        ],
    )(ea, we, be, deg)


def _mlpall(ea, we, be, w0, wg, bs):
    m = ea.shape[0]
    return pl.pallas_call(
        _mlpall_body,
        grid=(m // R,),
        in_specs=[_rows(R, 3), _whole((3, F0)), _whole((1, F0)),
                  _whole((F0, FH)), _whole((LAYERS - 1, FH, FH)),
                  _whole((LAYERS, FH))],
        out_specs=_whole((1, FH)),
        out_shape=jax.ShapeDtypeStruct((1, FH), jnp.float32),
    )(ea, we, be, w0, wg, bs)


def _mlp(a, w, b):
    m, f = a.shape
    fo = w.shape[1]
    return pl.pallas_call(
        _mlp_body,
        grid=(m // R,),
        in_specs=[_rows(R, f), _whole((f, fo)), _whole((1, fo))],
        out_specs=_rows(R, fo),
        out_shape=jax.ShapeDtypeStruct((m, fo), jnp.float32),
    )(a, w, b)


def _mix(a, ag, deg, w, b):
    f = a.shape[1]
    fo = w.shape[1]
    return pl.pallas_call(
        _mix_body,
        grid=(N // R,),
        in_specs=[_rows(R, f), _rows(R, f), _rows(R, 1),
                  _whole((f, fo)), _whole((1, fo))],
        out_specs=[_rows(R, fo)] + [_rows(R, fo // 4)] * 4,
        out_shape=([jax.ShapeDtypeStruct((N, fo), jnp.float32)]
                   + [jax.ShapeDtypeStruct((N, fo // 4), jnp.float32)] * 4),
    )(a, ag, deg, w, b)


def _colsum(a):
    m, f = a.shape
    return pl.pallas_call(
        _colsum_body,
        grid=(m // R,),
        in_specs=[_rows(R, f)],
        out_specs=_whole((1, f)),
        out_shape=jax.ShapeDtypeStruct((1, f), jnp.float32),
    )(a)


def _head(sums, w1, b1, w2, b2):
    return pl.pallas_call(
        _head_body,
        grid=(1,),
        in_specs=([_whole((1, FH))] * len(sums)
                  + [_whole((FH, 32)), _whole((1, 32)), _whole((32, 2)),
                     _whole((1, 2))]),
        out_specs=_whole((1, 2)),
        out_shape=jax.ShapeDtypeStruct((1, 2), jnp.float32),
    )(*sums, w1, b1, w2, b2)


def kernel(x, edge_index, edge_attr, batch, Wn, bn, We, be, Wg0, bg0, Wg, bg,
           W1, b1, W2, b2):
    src3 = edge_index[0].reshape(NTI, NWIN, WB)
    dst3 = edge_index[1].reshape(NTI, NWIN, WB)
    src4 = edge_index[0].reshape(NTI, 2 * NWIN, WB // 2)
    dst4 = edge_index[1].reshape(NTI, 2 * NWIN, WB // 2)

    ones_d = jnp.ones((NDEG,), jnp.float32)
    zeros0 = jnp.zeros((NP, F0 // 2), jnp.float32)
    zerosh = jnp.zeros((NP, FH // 4), jnp.float32)

    deg = _get_sc_degree()(dst3, ones_d)[:N].reshape(N, 1)

    wg0 = jnp.pad(Wg0, ((0, 0), (0, FH - Wg0.shape[1])))
    bg0p = jnp.pad(bg0, (0, FH - bg0.shape[0])).reshape(1, FH)
    wgs = [jnp.pad(Wg[i], ((0, FH - Wg.shape[1]), (0, FH - Wg.shape[2])))
           for i in range(Wg.shape[0])]
    bgs = [jnp.pad(bg[i], (0, FH - bg.shape[1])).reshape(1, FH)
           for i in range(Wg.shape[0])]
    w1p = jnp.pad(W1, ((0, FH - W1.shape[0]), (0, 0)))
    b1p = b1.reshape(1, 32)
    b2p = b2.reshape(1, 2)

    wep = jnp.pad(We, ((0, 0), (0, 0)))
    bep = be.reshape(1, F0)

    attr_lo, zlo, zhi = _embed_lo(edge_attr[:N], wep, bep, deg)
    wg_bf = jnp.stack(wgs[0:]).astype(jnp.bfloat16)
    bs_all = jnp.concatenate([bg0p] + bgs, axis=0)
    w0_bf = wg0.astype(jnp.bfloat16)
    shis = []
    splits = [(N, 47000), (57000, 37000), (94000, 33000), (127000, 33000)]
    for start, count in splits:
        shis.append(_mlpall(edge_attr[start:start + count], wep, bep,
                            w0_bf, wg_bf, bs_all))

    ws = [wg0] + wgs
    bs = [bg0p] + bgs
    ag = _make_sc_agg(F0 // 2, 1, WB, NWIN)(zlo, zhi, src3, dst3, zeros0)
    out = _mix(attr_lo, ag, deg, ws[0], bs[0])
    attr_lo, zq = out[0], list(out[1:])
    for l in range(1, len(ws)):
        ag = _make_sc_agg(FH // 4, 2, WB // 2, 2 * NWIN)(
            *zq, src4, dst4, zerosh)
        out = _mix(attr_lo, ag, deg, ws[l], bs[l])
        attr_lo, zq = out[0], list(out[1:])

    slo = _colsum(attr_lo)
    return _head([slo] + shis, w1p, b1p, W2, b2p)
